# initial kernel scaffold (unmeasured)
import jax
import jax.numpy as jnp
from jax import lax
from jax.experimental import pallas as pl
from jax.experimental.pallas import tpu as pltpu


def kernel(
    x,
):
    def body(*refs):
        pass

    out_shape = jax.ShapeDtypeStruct(..., jnp.float32)
    return pl.pallas_call(body, out_shape=out_shape)(...)


# baseline (device time: 29199 ns/iter reference)
import jax
import jax.numpy as jnp
from jax import lax
from jax.experimental import pallas as pl
from jax.experimental.pallas import tpu as pltpu


def kernel(x):
    m, n = x.shape

    def body(x_ref, out_ref, comm_ref, send_sem, recv_sem):
        my_x = lax.axis_index("x")
        my_y = lax.axis_index("y")
        my_z = lax.axis_index("z")
        partner = (1 - my_x, my_y, my_z)

        barrier_sem = pltpu.get_barrier_semaphore()
        pl.semaphore_signal(
            barrier_sem, inc=1,
            device_id=partner, device_id_type=pl.DeviceIdType.MESH,
        )
        pl.semaphore_wait(barrier_sem, 1)

        rdma = pltpu.make_async_remote_copy(
            src_ref=x_ref,
            dst_ref=comm_ref,
            send_sem=send_sem,
            recv_sem=recv_sem,
            device_id=partner,
            device_id_type=pl.DeviceIdType.MESH,
        )
        rdma.start()
        rdma.wait()

        out_ref[:, :] = x_ref[:, :] + comm_ref[:, :]

    return pl.pallas_call(
        body,
        out_shape=jax.ShapeDtypeStruct((m, n), x.dtype),
        in_specs=[pl.BlockSpec(memory_space=pltpu.VMEM)],
        out_specs=pl.BlockSpec(memory_space=pltpu.VMEM),
        scratch_shapes=[
            pltpu.VMEM((m, n), x.dtype),
            pltpu.SemaphoreType.DMA,
            pltpu.SemaphoreType.DMA,
        ],
        compiler_params=pltpu.CompilerParams(collective_id=0),
    )(x)


# device time: 21848 ns/iter; 1.3365x vs baseline; 1.3365x over previous
import jax
import jax.numpy as jnp
from jax import lax
from jax.experimental import pallas as pl
from jax.experimental.pallas import tpu as pltpu

K = 8


def kernel(x):
    m, n = x.shape
    half = m // 2
    rpc = half // K

    def body(x_ref, out_ref, commx_ref, sx_sems, rx_sems, sy_sems, ry_sems):
        my_x = lax.axis_index("x")
        my_y = lax.axis_index("y")
        my_z = lax.axis_index("z")
        partner_x = (1 - my_x, my_y, my_z)
        partner_y = (my_x, my_y + 1 - 2 * (my_y % 2), my_z)

        h_row = (my_y % 2) * half

        barrier_sem = pltpu.get_barrier_semaphore()
        for nbr in (partner_x, partner_y):
            pl.semaphore_signal(
                barrier_sem, inc=1,
                device_id=nbr, device_id_type=pl.DeviceIdType.MESH,
            )
        pl.semaphore_wait(barrier_sem, 2)

        rdma_x = []
        for k in range(K):
            r = pltpu.make_async_remote_copy(
                src_ref=x_ref.at[pl.ds(h_row + k * rpc, rpc), :],
                dst_ref=commx_ref.at[pl.ds(k * rpc, rpc), :],
                send_sem=sx_sems.at[k],
                recv_sem=rx_sems.at[k],
                device_id=partner_x,
                device_id_type=pl.DeviceIdType.MESH,
            )
            r.start()
            rdma_x.append(r)

        rdma_y = []
        for k in range(K):
            rdma_x[k].wait_recv()
            rows = pl.ds(h_row + k * rpc, rpc)
            out_ref[rows, :] = x_ref[rows, :] + commx_ref[pl.ds(k * rpc, rpc), :]
            r = pltpu.make_async_remote_copy(
                src_ref=out_ref.at[rows, :],
                dst_ref=out_ref.at[rows, :],
                send_sem=sy_sems.at[k],
                recv_sem=ry_sems.at[k],
                device_id=partner_y,
                device_id_type=pl.DeviceIdType.MESH,
            )
            r.start()
            rdma_y.append(r)

        for k in range(K):
            rdma_y[k].wait_recv()
        for k in range(K):
            rdma_x[k].wait_send()
            rdma_y[k].wait_send()

    return pl.pallas_call(
        body,
        out_shape=jax.ShapeDtypeStruct((m, n), x.dtype),
        in_specs=[pl.BlockSpec(memory_space=pltpu.VMEM)],
        out_specs=pl.BlockSpec(memory_space=pltpu.VMEM),
        scratch_shapes=[
            pltpu.VMEM((half, n), x.dtype),
            pltpu.SemaphoreType.DMA((K,)),
            pltpu.SemaphoreType.DMA((K,)),
            pltpu.SemaphoreType.DMA((K,)),
            pltpu.SemaphoreType.DMA((K,)),
        ],
        compiler_params=pltpu.CompilerParams(collective_id=0),
    )(x)


# device time: 21824 ns/iter; 1.3379x vs baseline; 1.0011x over previous
import jax
import jax.numpy as jnp
from jax import lax
from jax.experimental import pallas as pl
from jax.experimental.pallas import tpu as pltpu

K = 8


def kernel(x):
    m, n = x.shape
    half = m // 2
    rpc = half // K

    def body(
        x_ref, out_ref, commx_ref, sendy_ref, commy_ref,
        sx_sems, rx_sems, sy_sems, ry_sems,
    ):
        my_x = lax.axis_index("x")
        my_y = lax.axis_index("y")
        my_z = lax.axis_index("z")
        partner_x = (1 - my_x, my_y, my_z)
        partner_y = (my_x, my_y + 1 - 2 * (my_y % 2), my_z)

        h_row = (my_y % 2) * half

        barrier_sem = pltpu.get_barrier_semaphore()
        for nbr in (partner_x, partner_y):
            pl.semaphore_signal(
                barrier_sem, inc=1,
                device_id=nbr, device_id_type=pl.DeviceIdType.MESH,
            )
        pl.semaphore_wait(barrier_sem, 2)

        rdma_x = []
        for k in range(K):
            r = pltpu.make_async_remote_copy(
                src_ref=x_ref.at[pl.ds(h_row + k * rpc, rpc), :],
                dst_ref=commx_ref.at[pl.ds(k * rpc, rpc), :],
                send_sem=sx_sems.at[k],
                recv_sem=rx_sems.at[k],
                device_id=partner_x,
                device_id_type=pl.DeviceIdType.MESH,
            )
            r.start()
            rdma_x.append(r)

        rdma_y = []
        for k in range(K):
            rdma_x[k].wait_recv()
            chunk = pl.ds(k * rpc, rpc)
            s = x_ref[pl.ds(h_row + k * rpc, rpc), :] + commx_ref[chunk, :]
            sendy_ref[chunk, :] = s
            r = pltpu.make_async_remote_copy(
                src_ref=sendy_ref.at[chunk, :],
                dst_ref=commy_ref.at[chunk, :],
                send_sem=sy_sems.at[k],
                recv_sem=ry_sems.at[k],
                device_id=partner_y,
                device_id_type=pl.DeviceIdType.MESH,
            )
            r.start()
            rdma_y.append(r)
            out_ref[pl.ds(h_row + k * rpc, rpc), :] = s

        oh_row = half - h_row
        for k in range(K):
            rdma_y[k].wait_recv()
            out_ref[pl.ds(oh_row + k * rpc, rpc), :] = commy_ref[pl.ds(k * rpc, rpc), :]
        for k in range(K):
            rdma_x[k].wait_send()
            rdma_y[k].wait_send()

    return pl.pallas_call(
        body,
        out_shape=jax.ShapeDtypeStruct((m, n), x.dtype),
        in_specs=[pl.BlockSpec(memory_space=pltpu.VMEM)],
        out_specs=pl.BlockSpec(memory_space=pltpu.VMEM),
        scratch_shapes=[
            pltpu.VMEM((half, n), x.dtype),
            pltpu.VMEM((half, n), x.dtype),
            pltpu.VMEM((half, n), x.dtype),
            pltpu.SemaphoreType.DMA((K,)),
            pltpu.SemaphoreType.DMA((K,)),
            pltpu.SemaphoreType.DMA((K,)),
            pltpu.SemaphoreType.DMA((K,)),
        ],
        compiler_params=pltpu.CompilerParams(collective_id=0),
    )(x)


# device time: 20191 ns/iter; 1.4461x vs baseline; 1.0809x over previous
import jax
import jax.numpy as jnp
from jax import lax
from jax.experimental import pallas as pl
from jax.experimental.pallas import tpu as pltpu

K = 4


def kernel(x):
    m, n = x.shape
    qrows = m // 4
    rpc = qrows // K
    H = K // 2

    def body(
        x_ref, out_ref,
        commx_ref, sendq_ref, recvy_ref, recvz_ref, recvd_ref,
        sx, rx, sy, ry, sz, rz, sfy, rfy, sfz, rfz,
    ):
        my_x = lax.axis_index("x")
        my_y = lax.axis_index("y")
        my_z = lax.axis_index("z")
        py = my_y % 2
        pz = my_z % 2
        partner_x = (1 - my_x, my_y, my_z)
        partner_y = (my_x, my_y + 1 - 2 * py, my_z)
        partner_z = (my_x, my_y, my_z + 1 - 2 * pz)

        qm = 2 * py + pz
        qy = 2 * (1 - py) + pz
        qz = 2 * py + (1 - pz)
        qd = 2 * (1 - py) + (1 - pz)

        barrier_sem = pltpu.get_barrier_semaphore()
        for nbr in (partner_x, partner_y, partner_z):
            pl.semaphore_signal(
                barrier_sem, inc=1,
                device_id=nbr, device_id_type=pl.DeviceIdType.MESH,
            )
        pl.semaphore_wait(barrier_sem, 3)

        rdma_x = []
        for k in range(K):
            r = pltpu.make_async_remote_copy(
                src_ref=x_ref.at[pl.ds(qm * qrows + k * rpc, rpc), :],
                dst_ref=commx_ref.at[pl.ds(k * rpc, rpc), :],
                send_sem=sx.at[k],
                recv_sem=rx.at[k],
                device_id=partner_x,
                device_id_type=pl.DeviceIdType.MESH,
            )
            r.start()
            rdma_x.append(r)

        rdma_dir = []
        for k in range(K):
            rdma_x[k].wait_recv()
            chunk = pl.ds(k * rpc, rpc)
            s = x_ref[pl.ds(qm * qrows + k * rpc, rpc), :] + commx_ref[chunk, :]
            sendq_ref[chunk, :] = s
            for dev, dstbuf, ssem, rsem in (
                (partner_y, recvy_ref, sy, ry),
                (partner_z, recvz_ref, sz, rz),
            ):
                r = pltpu.make_async_remote_copy(
                    src_ref=sendq_ref.at[chunk, :],
                    dst_ref=dstbuf.at[chunk, :],
                    send_sem=ssem.at[k],
                    recv_sem=rsem.at[k],
                    device_id=dev,
                    device_id_type=pl.DeviceIdType.MESH,
                )
                r.start()
                rdma_dir.append(r)
            out_ref[pl.ds(qm * qrows + k * rpc, rpc), :] = s

        rdma_y = []
        rdma_fwd = []
        for k in range(K):
            chunk = pl.ds(k * rpc, rpc)

            ryd = pltpu.make_async_remote_copy(
                src_ref=sendq_ref.at[chunk, :], dst_ref=recvy_ref.at[chunk, :],
                send_sem=sy.at[k], recv_sem=ry.at[k],
                device_id=partner_y, device_id_type=pl.DeviceIdType.MESH,
            )
            ryd.wait_recv()
            out_ref[pl.ds(qy * qrows + k * rpc, rpc), :] = recvy_ref[chunk, :]
            if k < H:
                r = pltpu.make_async_remote_copy(
                    src_ref=recvy_ref.at[chunk, :],
                    dst_ref=recvd_ref.at[chunk, :],
                    send_sem=sfz.at[k],
                    recv_sem=rfz.at[k],
                    device_id=partner_z,
                    device_id_type=pl.DeviceIdType.MESH,
                )
                r.start()
                rdma_fwd.append(r)

            rzd = pltpu.make_async_remote_copy(
                src_ref=sendq_ref.at[chunk, :], dst_ref=recvz_ref.at[chunk, :],
                send_sem=sz.at[k], recv_sem=rz.at[k],
                device_id=partner_z, device_id_type=pl.DeviceIdType.MESH,
            )
            rzd.wait_recv()
            out_ref[pl.ds(qz * qrows + k * rpc, rpc), :] = recvz_ref[chunk, :]
            if k >= H:
                r = pltpu.make_async_remote_copy(
                    src_ref=recvz_ref.at[chunk, :],
                    dst_ref=recvd_ref.at[chunk, :],
                    send_sem=sfy.at[k],
                    recv_sem=rfy.at[k],
                    device_id=partner_y,
                    device_id_type=pl.DeviceIdType.MESH,
                )
                r.start()
                rdma_fwd.append(r)

        for k in range(K):
            chunk = pl.ds(k * rpc, rpc)
            if k < H:
                rsem, ssem, dev = rfz.at[k], sfz.at[k], partner_z
            else:
                rsem, ssem, dev = rfy.at[k], sfy.at[k], partner_y
            rd = pltpu.make_async_remote_copy(
                src_ref=recvd_ref.at[chunk, :], dst_ref=recvd_ref.at[chunk, :],
                send_sem=ssem, recv_sem=rsem,
                device_id=dev, device_id_type=pl.DeviceIdType.MESH,
            )
            rd.wait_recv()
            out_ref[pl.ds(qd * qrows + k * rpc, rpc), :] = recvd_ref[chunk, :]

        for r in rdma_x + rdma_dir + rdma_fwd:
            r.wait_send()

    return pl.pallas_call(
        body,
        out_shape=jax.ShapeDtypeStruct((m, n), x.dtype),
        in_specs=[pl.BlockSpec(memory_space=pltpu.VMEM)],
        out_specs=pl.BlockSpec(memory_space=pltpu.VMEM),
        scratch_shapes=[
            pltpu.VMEM((qrows, n), x.dtype),
            pltpu.VMEM((qrows, n), x.dtype),
            pltpu.VMEM((qrows, n), x.dtype),
            pltpu.VMEM((qrows, n), x.dtype),
            pltpu.VMEM((qrows, n), x.dtype),
            pltpu.SemaphoreType.DMA((K,)),
            pltpu.SemaphoreType.DMA((K,)),
            pltpu.SemaphoreType.DMA((K,)),
            pltpu.SemaphoreType.DMA((K,)),
            pltpu.SemaphoreType.DMA((K,)),
            pltpu.SemaphoreType.DMA((K,)),
            pltpu.SemaphoreType.DMA((K,)),
            pltpu.SemaphoreType.DMA((K,)),
            pltpu.SemaphoreType.DMA((K,)),
            pltpu.SemaphoreType.DMA((K,)),
        ],
        compiler_params=pltpu.CompilerParams(collective_id=0),
    )(x)


# device time: 19855 ns/iter; 1.4706x vs baseline; 1.0169x over previous
import jax
import jax.numpy as jnp
from jax import lax
from jax.experimental import pallas as pl
from jax.experimental.pallas import tpu as pltpu

K = 8


def kernel(x):
    m, n = x.shape
    qrows = m // 4
    rpc = qrows // K
    H = K // 2

    def body(
        x_ref, out_ref,
        commx_ref, sendq_ref, recvy_ref, recvz_ref, recvd_ref,
        sx, rx, sy, ry, sz, rz, sfy, rfy, sfz, rfz,
    ):
        my_x = lax.axis_index("x")
        my_y = lax.axis_index("y")
        my_z = lax.axis_index("z")
        py = my_y % 2
        pz = my_z % 2
        partner_x = (1 - my_x, my_y, my_z)
        partner_y = (my_x, my_y + 1 - 2 * py, my_z)
        partner_z = (my_x, my_y, my_z + 1 - 2 * pz)

        qm = 2 * py + pz
        qy = 2 * (1 - py) + pz
        qz = 2 * py + (1 - pz)
        qd = 2 * (1 - py) + (1 - pz)

        barrier_sem = pltpu.get_barrier_semaphore()
        for nbr in (partner_x, partner_y, partner_z):
            pl.semaphore_signal(
                barrier_sem, inc=1,
                device_id=nbr, device_id_type=pl.DeviceIdType.MESH,
            )
        pl.semaphore_wait(barrier_sem, 3)

        rdma_x = []
        for k in range(K):
            r = pltpu.make_async_remote_copy(
                src_ref=x_ref.at[pl.ds(qm * qrows + k * rpc, rpc), :],
                dst_ref=commx_ref.at[pl.ds(k * rpc, rpc), :],
                send_sem=sx.at[k],
                recv_sem=rx.at[k],
                device_id=partner_x,
                device_id_type=pl.DeviceIdType.MESH,
            )
            r.start()
            rdma_x.append(r)

        rdma_dir = []
        for k in range(K):
            rdma_x[k].wait_recv()
            chunk = pl.ds(k * rpc, rpc)
            s = x_ref[pl.ds(qm * qrows + k * rpc, rpc), :] + commx_ref[chunk, :]
            sendq_ref[chunk, :] = s
            for dev, dstbuf, ssem, rsem in (
                (partner_y, recvy_ref, sy, ry),
                (partner_z, recvz_ref, sz, rz),
            ):
                r = pltpu.make_async_remote_copy(
                    src_ref=sendq_ref.at[chunk, :],
                    dst_ref=dstbuf.at[chunk, :],
                    send_sem=ssem.at[k],
                    recv_sem=rsem.at[k],
                    device_id=dev,
                    device_id_type=pl.DeviceIdType.MESH,
                )
                r.start()
                rdma_dir.append(r)
            out_ref[pl.ds(qm * qrows + k * rpc, rpc), :] = s

        rdma_y = []
        rdma_fwd = []
        for k in range(K):
            chunk = pl.ds(k * rpc, rpc)

            ryd = pltpu.make_async_remote_copy(
                src_ref=sendq_ref.at[chunk, :], dst_ref=recvy_ref.at[chunk, :],
                send_sem=sy.at[k], recv_sem=ry.at[k],
                device_id=partner_y, device_id_type=pl.DeviceIdType.MESH,
            )
            ryd.wait_recv()
            out_ref[pl.ds(qy * qrows + k * rpc, rpc), :] = recvy_ref[chunk, :]
            if k < H:
                r = pltpu.make_async_remote_copy(
                    src_ref=recvy_ref.at[chunk, :],
                    dst_ref=recvd_ref.at[chunk, :],
                    send_sem=sfz.at[k],
                    recv_sem=rfz.at[k],
                    device_id=partner_z,
                    device_id_type=pl.DeviceIdType.MESH,
                )
                r.start()
                rdma_fwd.append(r)

            rzd = pltpu.make_async_remote_copy(
                src_ref=sendq_ref.at[chunk, :], dst_ref=recvz_ref.at[chunk, :],
                send_sem=sz.at[k], recv_sem=rz.at[k],
                device_id=partner_z, device_id_type=pl.DeviceIdType.MESH,
            )
            rzd.wait_recv()
            out_ref[pl.ds(qz * qrows + k * rpc, rpc), :] = recvz_ref[chunk, :]
            if k >= H:
                r = pltpu.make_async_remote_copy(
                    src_ref=recvz_ref.at[chunk, :],
                    dst_ref=recvd_ref.at[chunk, :],
                    send_sem=sfy.at[k],
                    recv_sem=rfy.at[k],
                    device_id=partner_y,
                    device_id_type=pl.DeviceIdType.MESH,
                )
                r.start()
                rdma_fwd.append(r)

        for k in range(K):
            chunk = pl.ds(k * rpc, rpc)
            if k < H:
                rsem, ssem, dev = rfz.at[k], sfz.at[k], partner_z
            else:
                rsem, ssem, dev = rfy.at[k], sfy.at[k], partner_y
            rd = pltpu.make_async_remote_copy(
                src_ref=recvd_ref.at[chunk, :], dst_ref=recvd_ref.at[chunk, :],
                send_sem=ssem, recv_sem=rsem,
                device_id=dev, device_id_type=pl.DeviceIdType.MESH,
            )
            rd.wait_recv()
            out_ref[pl.ds(qd * qrows + k * rpc, rpc), :] = recvd_ref[chunk, :]

        for r in rdma_x + rdma_dir + rdma_fwd:
            r.wait_send()

    return pl.pallas_call(
        body,
        out_shape=jax.ShapeDtypeStruct((m, n), x.dtype),
        in_specs=[pl.BlockSpec(memory_space=pltpu.VMEM)],
        out_specs=pl.BlockSpec(memory_space=pltpu.VMEM),
        scratch_shapes=[
            pltpu.VMEM((qrows, n), x.dtype),
            pltpu.VMEM((qrows, n), x.dtype),
            pltpu.VMEM((qrows, n), x.dtype),
            pltpu.VMEM((qrows, n), x.dtype),
            pltpu.VMEM((qrows, n), x.dtype),
            pltpu.SemaphoreType.DMA((K,)),
            pltpu.SemaphoreType.DMA((K,)),
            pltpu.SemaphoreType.DMA((K,)),
            pltpu.SemaphoreType.DMA((K,)),
            pltpu.SemaphoreType.DMA((K,)),
            pltpu.SemaphoreType.DMA((K,)),
            pltpu.SemaphoreType.DMA((K,)),
            pltpu.SemaphoreType.DMA((K,)),
            pltpu.SemaphoreType.DMA((K,)),
            pltpu.SemaphoreType.DMA((K,)),
        ],
        compiler_params=pltpu.CompilerParams(collective_id=0),
    )(x)


# device time: 19155 ns/iter; 1.5244x vs baseline; 1.0365x over previous
import jax
import jax.numpy as jnp
from jax import lax
from jax.experimental import pallas as pl
from jax.experimental.pallas import tpu as pltpu

K = 8
XD = 2
FZ = (K - XD) // 2


def kernel(x):
    m, n = x.shape
    qrows = m // 4
    rpc = qrows // K

    def body(
        x_ref, out_ref,
        commx_ref, commxd_ref, sendq_ref, recvy_ref, recvz_ref, recvd_ref,
        sx, rx, sy, ry, sz, rz, sfy, rfy, sfz, rfz,
    ):
        my_x = lax.axis_index("x")
        my_y = lax.axis_index("y")
        my_z = lax.axis_index("z")
        py = my_y % 2
        pz = my_z % 2
        partner_x = (1 - my_x, my_y, my_z)
        partner_y = (my_x, my_y + 1 - 2 * py, my_z)
        partner_z = (my_x, my_y, my_z + 1 - 2 * pz)

        qm = 2 * py + pz
        qy = 2 * (1 - py) + pz
        qz = 2 * py + (1 - pz)
        qd = 2 * (1 - py) + (1 - pz)

        barrier_sem = pltpu.get_barrier_semaphore()
        for nbr in (partner_x, partner_y, partner_z):
            pl.semaphore_signal(
                barrier_sem, inc=1,
                device_id=nbr, device_id_type=pl.DeviceIdType.MESH,
            )
        pl.semaphore_wait(barrier_sem, 3)

        rdma_x = []
        for k in range(K):
            r = pltpu.make_async_remote_copy(
                src_ref=x_ref.at[pl.ds(qm * qrows + k * rpc, rpc), :],
                dst_ref=commx_ref.at[pl.ds(k * rpc, rpc), :],
                send_sem=sx.at[k],
                recv_sem=rx.at[k],
                device_id=partner_x,
                device_id_type=pl.DeviceIdType.MESH,
            )
            r.start()
            rdma_x.append(r)
        for j in range(XD):
            k = K - XD + j
            r = pltpu.make_async_remote_copy(
                src_ref=x_ref.at[pl.ds(qd * qrows + k * rpc, rpc), :],
                dst_ref=commxd_ref.at[pl.ds(j * rpc, rpc), :],
                send_sem=sx.at[K + j],
                recv_sem=rx.at[K + j],
                device_id=partner_x,
                device_id_type=pl.DeviceIdType.MESH,
            )
            r.start()
            rdma_x.append(r)

        rdma_dir = []
        for k in range(K):
            rdma_x[k].wait_recv()
            chunk = pl.ds(k * rpc, rpc)
            s = x_ref[pl.ds(qm * qrows + k * rpc, rpc), :] + commx_ref[chunk, :]
            sendq_ref[chunk, :] = s
            for dev, dstbuf, ssem, rsem in (
                (partner_y, recvy_ref, sy, ry),
                (partner_z, recvz_ref, sz, rz),
            ):
                r = pltpu.make_async_remote_copy(
                    src_ref=sendq_ref.at[chunk, :],
                    dst_ref=dstbuf.at[chunk, :],
                    send_sem=ssem.at[k],
                    recv_sem=rsem.at[k],
                    device_id=dev,
                    device_id_type=pl.DeviceIdType.MESH,
                )
                r.start()
                rdma_dir.append(r)
            out_ref[pl.ds(qm * qrows + k * rpc, rpc), :] = s

        rdma_fwd = []
        for k in range(K):
            chunk = pl.ds(k * rpc, rpc)

            ryd = pltpu.make_async_remote_copy(
                src_ref=sendq_ref.at[chunk, :], dst_ref=recvy_ref.at[chunk, :],
                send_sem=sy.at[k], recv_sem=ry.at[k],
                device_id=partner_y, device_id_type=pl.DeviceIdType.MESH,
            )
            ryd.wait_recv()
            out_ref[pl.ds(qy * qrows + k * rpc, rpc), :] = recvy_ref[chunk, :]
            if k < FZ:
                r = pltpu.make_async_remote_copy(
                    src_ref=recvy_ref.at[chunk, :],
                    dst_ref=recvd_ref.at[chunk, :],
                    send_sem=sfz.at[k],
                    recv_sem=rfz.at[k],
                    device_id=partner_z,
                    device_id_type=pl.DeviceIdType.MESH,
                )
                r.start()
                rdma_fwd.append(r)

            rzd = pltpu.make_async_remote_copy(
                src_ref=sendq_ref.at[chunk, :], dst_ref=recvz_ref.at[chunk, :],
                send_sem=sz.at[k], recv_sem=rz.at[k],
                device_id=partner_z, device_id_type=pl.DeviceIdType.MESH,
            )
            rzd.wait_recv()
            out_ref[pl.ds(qz * qrows + k * rpc, rpc), :] = recvz_ref[chunk, :]
            if FZ <= k < K - XD:
                r = pltpu.make_async_remote_copy(
                    src_ref=recvz_ref.at[chunk, :],
                    dst_ref=recvd_ref.at[chunk, :],
                    send_sem=sfy.at[k],
                    recv_sem=rfy.at[k],
                    device_id=partner_y,
                    device_id_type=pl.DeviceIdType.MESH,
                )
                r.start()
                rdma_fwd.append(r)

        for j in range(XD):
            k = K - XD + j
            rdma_x[K + j].wait_recv()
            out_ref[pl.ds(qd * qrows + k * rpc, rpc), :] = (
                x_ref[pl.ds(qd * qrows + k * rpc, rpc), :]
                + commxd_ref[pl.ds(j * rpc, rpc), :]
            )

        for k in range(K - XD):
            chunk = pl.ds(k * rpc, rpc)
            if k < FZ:
                rsem, ssem, dev = rfz.at[k], sfz.at[k], partner_z
            else:
                rsem, ssem, dev = rfy.at[k], sfy.at[k], partner_y
            rd = pltpu.make_async_remote_copy(
                src_ref=recvd_ref.at[chunk, :], dst_ref=recvd_ref.at[chunk, :],
                send_sem=ssem, recv_sem=rsem,
                device_id=dev, device_id_type=pl.DeviceIdType.MESH,
            )
            rd.wait_recv()
            out_ref[pl.ds(qd * qrows + k * rpc, rpc), :] = recvd_ref[chunk, :]

        for r in rdma_x + rdma_dir + rdma_fwd:
            r.wait_send()

    return pl.pallas_call(
        body,
        out_shape=jax.ShapeDtypeStruct((m, n), x.dtype),
        in_specs=[pl.BlockSpec(memory_space=pltpu.VMEM)],
        out_specs=pl.BlockSpec(memory_space=pltpu.VMEM),
        scratch_shapes=[
            pltpu.VMEM((qrows, n), x.dtype),
            pltpu.VMEM((XD * rpc, n), x.dtype),
            pltpu.VMEM((qrows, n), x.dtype),
            pltpu.VMEM((qrows, n), x.dtype),
            pltpu.VMEM((qrows, n), x.dtype),
            pltpu.VMEM((qrows, n), x.dtype),
            pltpu.SemaphoreType.DMA((K + XD,)),
            pltpu.SemaphoreType.DMA((K + XD,)),
            pltpu.SemaphoreType.DMA((K,)),
            pltpu.SemaphoreType.DMA((K,)),
            pltpu.SemaphoreType.DMA((K,)),
            pltpu.SemaphoreType.DMA((K,)),
            pltpu.SemaphoreType.DMA((K,)),
            pltpu.SemaphoreType.DMA((K,)),
            pltpu.SemaphoreType.DMA((K,)),
            pltpu.SemaphoreType.DMA((K,)),
        ],
        compiler_params=pltpu.CompilerParams(collective_id=0),
    )(x)
